# baseline (speedup 1.0000x reference)
import functools

import jax
import jax.numpy as jnp
from jax import lax
from jax.experimental import pallas as pl
from jax.experimental.pallas import tpu as pltpu
from jax.experimental.pallas import tpu_sc as plsc

NC = 2
NS = 16
L = 16
CH = 128


@functools.lru_cache(maxsize=None)
def _build(B: int, D: int, V: int):
    NW = NC * NS
    b_per_w = B // NW
    n_ch = b_per_w // CH
    n_grp = b_per_w // L
    VH = V // 2

    mesh = plsc.VectorSubcoreMesh(
        core_axis_name="c", subcore_axis_name="s",
        num_cores=NC, num_subcores=NS)

    @functools.partial(
        pl.kernel,
        out_type=jax.ShapeDtypeStruct((B,), jnp.float32),
        mesh=mesh,
        scratch_types=[
            pltpu.VMEM((n_ch, CH), jnp.int32),
            pltpu.VMEM((n_ch, CH), jnp.int32),
            pltpu.VMEM((n_ch, CH), jnp.int32),
            pltpu.VMEM((n_ch, CH), jnp.int32),
            pltpu.VMEM((b_per_w,), jnp.int32),
            pltpu.VMEM((b_per_w,), jnp.int32),
            pltpu.VMEM((2 * CH, D), jnp.float32),
            pltpu.VMEM((2 * CH, D), jnp.float32),
            pltpu.VMEM((2 * CH, D), jnp.float32),
            pltpu.VMEM((2 * CH, D), jnp.float32),
            pltpu.VMEM((b_per_w,), jnp.float32),
            pltpu.SemaphoreType.DMA,
        ],
        compiler_params=pltpu.CompilerParams(
            needs_layout_passes=False, use_tc_tiling_on_sc=False),
    )
    def dis_kernel(src_hbm, dst_hbm, tlo_hbm, thi_hbm, out_hbm,
                   slo, dlo, shi, dhi, sraw, draw_,
                   srows_lo, drows_lo, srows_hi, drows_hi, obuf, sem):
        wid = lax.axis_index("s") * NC + lax.axis_index("c")
        base_ch = wid * n_ch

        pltpu.sync_copy(src_hbm.at[pl.ds(base_ch, n_ch)], slo)
        pltpu.sync_copy(dst_hbm.at[pl.ds(base_ch, n_ch)], dlo)

        def prep(c, carry):
            for k in range(CH // L):
                sl2 = pl.ds(k * L, L)
                slf = pl.ds(c * CH + k * L, L)
                vs = slo[c, sl2]
                vd = dlo[c, sl2]
                sraw[slf] = vs
                draw_[slf] = vd
                shi[c, sl2] = jnp.minimum(jnp.maximum(vs - VH, 0), VH - 1)
                dhi[c, sl2] = jnp.minimum(jnp.maximum(vd - VH, 0), VH - 1)
                slo[c, sl2] = jnp.minimum(vs, VH - 1)
                dlo[c, sl2] = jnp.minimum(vd, VH - 1)
            return carry

        lax.fori_loop(0, n_ch, prep, 0)

        lane_iota = lax.iota(jnp.int32, L)

        def fire(c):
            buf_sl = pl.ds((c % 2) * CH, CH)
            return [
                pltpu.async_copy(tlo_hbm.at[slo.at[c]],
                                 srows_lo.at[buf_sl], sem),
                pltpu.async_copy(tlo_hbm.at[dlo.at[c]],
                                 drows_lo.at[buf_sl], sem),
                pltpu.async_copy(thi_hbm.at[shi.at[c]],
                                 srows_hi.at[buf_sl], sem),
                pltpu.async_copy(thi_hbm.at[dhi.at[c]],
                                 drows_hi.at[buf_sl], sem),
            ]

        def compute(c):
            buf0 = (c % 2) * CH

            def group(g, carry):
                lanes = buf0 + g * L + lane_iota
                sl = pl.ds(c * CH + g * L, L)
                ms = sraw[sl] < VH
                md = draw_[sl] < VH
                acc = jnp.zeros((L,), jnp.float32)
                for d in range(D):
                    col = jnp.full((L,), d, jnp.int32)
                    s_lo = plsc.load_gather(srows_lo, [lanes, col])
                    s_hi = plsc.load_gather(srows_hi, [lanes, col])
                    t_lo = plsc.load_gather(drows_lo, [lanes, col])
                    t_hi = plsc.load_gather(drows_hi, [lanes, col])
                    df = (jnp.where(ms, s_lo, s_hi)
                          - jnp.where(md, t_lo, t_hi))
                    acc = acc + df * df
                x = acc + jnp.float32(1e-12)
                i = plsc.bitcast(x, jnp.int32)
                r = plsc.bitcast(jnp.int32(0x5F3759DF) - (i >> 1),
                                 jnp.float32)
                half_x = jnp.float32(0.5) * x
                for _ in range(3):
                    r = r * (jnp.float32(1.5) - half_x * r * r)
                obuf[pl.ds(c * CH + g * L, L)] = x * r
                return carry

            lax.fori_loop(0, CH // L, group, 0)

        inflight = fire(0)
        for c in range(n_ch):
            nxt = fire(c + 1) if c + 1 < n_ch else ()
            for cp in inflight:
                cp.wait()
            compute(c)
            inflight = nxt

        pltpu.sync_copy(obuf, out_hbm.at[pl.ds(wid * b_per_w, b_per_w)])

    return dis_kernel


def kernel(input_triplet, table):
    B = input_triplet.shape[0]
    V, D = table.shape
    src = input_triplet[:, 0].astype(jnp.int32).reshape(B // CH, CH)
    dst = input_triplet[:, 1].astype(jnp.int32).reshape(B // CH, CH)
    tlo = table[: V // 2]
    thi = table[V // 2:]
    return _build(B, D, V)(src, dst, tlo, thi)


# speedup vs baseline: 1.9608x; 1.9608x over previous
import functools

import jax
import jax.numpy as jnp
from jax import lax
from jax.experimental import pallas as pl
from jax.experimental.pallas import tpu as pltpu
from jax.experimental.pallas import tpu_sc as plsc

NC = 2
NS = 16
L = 16
CH = 128


@functools.lru_cache(maxsize=None)
def _build(B: int, D: int):
    NW = NC * NS
    b_per_w = B // NW
    n_ch = b_per_w // CH
    n_grp = b_per_w // L

    mesh = plsc.VectorSubcoreMesh(
        core_axis_name="c", subcore_axis_name="s",
        num_cores=NC, num_subcores=NS)

    @functools.partial(
        pl.kernel,
        out_type=jax.ShapeDtypeStruct((B,), jnp.float32),
        mesh=mesh,
        scratch_types=[
            pltpu.VMEM((n_ch, CH), jnp.int32),
            pltpu.VMEM((n_ch, CH), jnp.int32),
            pltpu.VMEM((b_per_w, D), jnp.float32),
            pltpu.VMEM((b_per_w, D), jnp.float32),
            pltpu.VMEM((b_per_w,), jnp.float32),
            pltpu.SemaphoreType.DMA,
        ],
        compiler_params=pltpu.CompilerParams(
            needs_layout_passes=False, use_tc_tiling_on_sc=False),
    )
    def dis_kernel(src_hbm, dst_hbm, table_hbm, out_hbm,
                   sidx, didx, srows, drows, obuf, sem):
        wid = lax.axis_index("s") * NC + lax.axis_index("c")
        base_ch = wid * n_ch

        pltpu.sync_copy(src_hbm.at[pl.ds(base_ch, n_ch)], sidx)
        pltpu.sync_copy(dst_hbm.at[pl.ds(base_ch, n_ch)], didx)

        copies = []
        for c in range(n_ch):
            copies.append(pltpu.async_copy(
                table_hbm.at[sidx.at[c]],
                srows.at[pl.ds(c * CH, CH)], sem))
            copies.append(pltpu.async_copy(
                table_hbm.at[didx.at[c]],
                drows.at[pl.ds(c * CH, CH)], sem))
        for cp in copies:
            cp.wait()

        lane_iota = lax.iota(jnp.int32, L)

        def group(g, carry):
            lanes = g * L + lane_iota
            acc = jnp.zeros((L,), jnp.float32)
            for d in range(D):
                col = jnp.full((L,), d, jnp.int32)
                s = plsc.load_gather(srows, [lanes, col])
                t = plsc.load_gather(drows, [lanes, col])
                df = s - t
                acc = acc + df * df
            x = acc + jnp.float32(1e-12)
            i = plsc.bitcast(x, jnp.int32)
            r = plsc.bitcast(jnp.int32(0x5F3759DF) - (i >> 1), jnp.float32)
            half_x = jnp.float32(0.5) * x
            for _ in range(3):
                r = r * (jnp.float32(1.5) - half_x * r * r)
            obuf[pl.ds(g * L, L)] = x * r
            return carry

        lax.fori_loop(0, n_grp, group, 0)
        pltpu.sync_copy(obuf, out_hbm.at[pl.ds(wid * b_per_w, b_per_w)])

    return dis_kernel


def kernel(input_triplet, table):
    B = input_triplet.shape[0]
    D = table.shape[1]
    src = input_triplet[:, 0].astype(jnp.int32).reshape(B // CH, CH)
    dst = input_triplet[:, 1].astype(jnp.int32).reshape(B // CH, CH)
    return _build(B, D)(src, dst, table)


# speedup vs baseline: 5.2081x; 2.6561x over previous
import functools

import jax
import jax.numpy as jnp
from jax import lax
from jax.experimental import pallas as pl
from jax.experimental.pallas import tpu as pltpu
from jax.experimental.pallas import tpu_sc as plsc

NC = 2
NS = 16
L = 16

SLICE = 62464
MAINT = NS * SLICE
TAILW = 640


def _mesh():
    return plsc.VectorSubcoreMesh(
        core_axis_name="c", subcore_axis_name="s",
        num_cores=NC, num_subcores=NS)


@functools.lru_cache(maxsize=None)
def _build_main(B: int, D: int, V: int):
    d_per_sc = D // NC
    b_per_t = B // NS
    n_grp = b_per_t // L

    @functools.partial(
        pl.kernel,
        out_type=jax.ShapeDtypeStruct((NC, B), jnp.float32),
        mesh=_mesh(),
        scratch_types=[
            pltpu.VMEM_SHARED((MAINT,), jnp.float32),
            pltpu.VMEM((b_per_t,), jnp.int32),
            pltpu.VMEM((b_per_t,), jnp.int32),
            pltpu.VMEM((b_per_t,), jnp.int32),
            pltpu.VMEM((b_per_t,), jnp.int32),
            pltpu.VMEM((b_per_t,), jnp.float32),
            pltpu.VMEM((b_per_t,), jnp.float32),
            pltpu.VMEM((d_per_sc, TAILW), jnp.float32),
            pltpu.VMEM((b_per_t,), jnp.float32),
            pltpu.SemaphoreType.DMA,
            pltpu.SemaphoreType.DMA,
        ],
        compiler_params=pltpu.CompilerParams(needs_layout_passes=False),
    )
    def main_kernel(src_hbm, dst_hbm, tabT_hbm, tail_hbm, part_hbm,
                    shared, scl, dcl, stb, dtb,
                    svals, dvals, tail_v, acc, sem_s, sem_g):
        cid = lax.axis_index("c")
        tid = lax.axis_index("s")
        sc_d0 = cid * d_per_sc

        pltpu.sync_copy(src_hbm.at[pl.ds(tid * b_per_t, b_per_t)], scl)
        pltpu.sync_copy(dst_hbm.at[pl.ds(tid * b_per_t, b_per_t)], dcl)
        pltpu.sync_copy(tail_hbm.at[pl.ds(sc_d0, d_per_sc)], tail_v)

        zero16 = jnp.zeros((L,), jnp.float32)

        def prep(g, carry):
            sl = pl.ds(g * L, L)
            vs = scl[sl]
            vd = dcl[sl]
            stb[sl] = jnp.maximum(vs - (MAINT - 1), 0)
            dtb[sl] = jnp.maximum(vd - (MAINT - 1), 0)
            scl[sl] = jnp.minimum(vs, MAINT - 1)
            dcl[sl] = jnp.minimum(vd, MAINT - 1)
            acc[sl] = zero16
            return carry

        lax.fori_loop(0, n_grp, prep, 0)

        def column(j, carry):
            d = sc_d0 + j
            plsc.subcore_barrier()
            pltpu.async_copy(
                tabT_hbm.at[d, pl.ds(tid * SLICE, SLICE)],
                shared.at[pl.ds(tid * SLICE, SLICE)], sem_s).wait()
            plsc.subcore_barrier()
            cps = [pltpu.async_copy(shared.at[scl], svals, sem_g),
                   pltpu.async_copy(shared.at[dcl], dvals, sem_g)]
            for cp in cps:
                cp.wait()

            dsplat = jnp.broadcast_to(j, (L,)).astype(jnp.int32)

            def group(g, c2):
                sl = pl.ds(g * L, L)
                sv = svals[sl]
                dv = dvals[sl]
                tbs = stb[sl]
                tbd = dtb[sl]
                ts = plsc.load_gather(
                    tail_v, [dsplat, jnp.maximum(tbs - 1, 0)])
                td = plsc.load_gather(
                    tail_v, [dsplat, jnp.maximum(tbd - 1, 0)])
                s_fin = jnp.where(tbs > 0, ts, sv)
                d_fin = jnp.where(tbd > 0, td, dv)
                df = s_fin - d_fin
                acc[sl] = acc[sl] + df * df
                return c2

            lax.fori_loop(0, n_grp, group, 0)
            return carry

        lax.fori_loop(0, d_per_sc, column, 0)

        pltpu.sync_copy(acc, part_hbm.at[cid, pl.ds(tid * b_per_t, b_per_t)])

    return main_kernel


@functools.lru_cache(maxsize=None)
def _build_combine(B: int):
    NW = NC * NS
    b_per_w = B // NW

    @functools.partial(
        pl.kernel,
        out_type=jax.ShapeDtypeStruct((B,), jnp.float32),
        mesh=_mesh(),
        scratch_types=[
            pltpu.VMEM((b_per_w,), jnp.float32),
            pltpu.VMEM((b_per_w,), jnp.float32),
            pltpu.VMEM((b_per_w,), jnp.float32),
        ],
        compiler_params=pltpu.CompilerParams(needs_layout_passes=False),
    )
    def combine_kernel(part_hbm, out_hbm, a0, a1, ob):
        wid = lax.axis_index("s") * NC + lax.axis_index("c")
        base = wid * b_per_w
        pltpu.sync_copy(part_hbm.at[0, pl.ds(base, b_per_w)], a0)
        pltpu.sync_copy(part_hbm.at[1, pl.ds(base, b_per_w)], a1)

        def group(i, carry):
            sl = pl.ds(i * L, L)
            x = a0[sl] + a1[sl] + jnp.float32(1e-12)
            iv = plsc.bitcast(x, jnp.int32)
            r = plsc.bitcast(jnp.int32(0x5F3759DF) - (iv >> 1), jnp.float32)
            half_x = jnp.float32(0.5) * x
            for _ in range(3):
                r = r * (jnp.float32(1.5) - half_x * r * r)
            ob[sl] = x * r
            return carry

        lax.fori_loop(0, b_per_w // L, group, 0)
        pltpu.sync_copy(ob, out_hbm.at[pl.ds(base, b_per_w)])

    return combine_kernel


def kernel(input_triplet, table):
    B = input_triplet.shape[0]
    V, D = table.shape
    src = input_triplet[:, 0].astype(jnp.int32)
    dst = input_triplet[:, 1].astype(jnp.int32)
    tabT = table.T
    tail = jnp.pad(table[MAINT:, :].T, ((0, 0), (0, TAILW - (V - MAINT))))
    part = _build_main(B, D, V)(src, dst, tabT, tail)
    return _build_combine(B)(part)


# speedup vs baseline: 5.5080x; 1.0576x over previous
import functools

import jax
import jax.numpy as jnp
from jax import lax
from jax.experimental import pallas as pl
from jax.experimental.pallas import tpu as pltpu
from jax.experimental.pallas import tpu_sc as plsc

NC = 2
NS = 16
L = 16

SLICE = 62464
MAINT = NS * SLICE
TAILW = 640


def _mesh():
    return plsc.VectorSubcoreMesh(
        core_axis_name="c", subcore_axis_name="s",
        num_cores=NC, num_subcores=NS)


@functools.lru_cache(maxsize=None)
def _build_main(B: int, D: int, V: int):
    d_per_sc = D // NC
    b_per_t = B // NS
    n_grp = b_per_t // L

    @functools.partial(
        pl.kernel,
        out_type=jax.ShapeDtypeStruct((NC, B), jnp.float32),
        mesh=_mesh(),
        scratch_types=[
            pltpu.VMEM_SHARED((MAINT,), jnp.float32),
            pltpu.VMEM((b_per_t,), jnp.int32),
            pltpu.VMEM((b_per_t,), jnp.int32),
            pltpu.VMEM((b_per_t,), jnp.int32),
            pltpu.VMEM((b_per_t,), jnp.int32),
            pltpu.VMEM((b_per_t,), jnp.float32),
            pltpu.VMEM((b_per_t,), jnp.float32),
            pltpu.VMEM((d_per_sc, TAILW), jnp.float32),
            pltpu.VMEM((b_per_t,), jnp.float32),
            pltpu.SemaphoreType.DMA,
            pltpu.SemaphoreType.DMA,
        ],
        compiler_params=pltpu.CompilerParams(needs_layout_passes=False),
    )
    def main_kernel(src_hbm, dst_hbm, tabT_hbm, tail_hbm, part_hbm,
                    shared, scl, dcl, stb, dtb,
                    svals, dvals, tail_v, acc, sem_s, sem_g):
        cid = lax.axis_index("c")
        tid = lax.axis_index("s")
        sc_d0 = cid * d_per_sc

        pltpu.sync_copy(src_hbm.at[pl.ds(tid * b_per_t, b_per_t)], scl)
        pltpu.sync_copy(dst_hbm.at[pl.ds(tid * b_per_t, b_per_t)], dcl)
        pltpu.sync_copy(tail_hbm.at[pl.ds(sc_d0, d_per_sc)], tail_v)

        zero16 = jnp.zeros((L,), jnp.float32)

        def prep(g, carry):
            sl = pl.ds(g * L, L)
            vs = scl[sl]
            vd = dcl[sl]
            stb[sl] = jnp.maximum(vs - (MAINT - 1), 0)
            dtb[sl] = jnp.maximum(vd - (MAINT - 1), 0)
            scl[sl] = jnp.minimum(vs, MAINT - 1)
            dcl[sl] = jnp.minimum(vd, MAINT - 1)
            acc[sl] = zero16
            return carry

        lax.fori_loop(0, n_grp, prep, 0)

        def fire_stage(j):
            d = sc_d0 + jnp.minimum(j, d_per_sc - 1)
            return pltpu.async_copy(
                tabT_hbm.at[d, pl.ds(tid * SLICE, SLICE)],
                shared.at[pl.ds(tid * SLICE, SLICE)], sem_s)

        def wait_stage():
            pltpu.make_async_copy(
                tabT_hbm.at[sc_d0, pl.ds(tid * SLICE, SLICE)],
                shared.at[pl.ds(tid * SLICE, SLICE)], sem_s).wait()

        fire_stage(0)

        def column(j, carry):
            wait_stage()
            plsc.subcore_barrier()
            cps = [pltpu.async_copy(shared.at[scl], svals, sem_g),
                   pltpu.async_copy(shared.at[dcl], dvals, sem_g)]
            for cp in cps:
                cp.wait()
            plsc.subcore_barrier()
            fire_stage(j + 1)

            dsplat = jnp.broadcast_to(j, (L,)).astype(jnp.int32)

            def group(g, c2):
                sl = pl.ds(g * L, L)
                sv = svals[sl]
                dv = dvals[sl]
                tbs = stb[sl]
                tbd = dtb[sl]
                ts = plsc.load_gather(
                    tail_v, [dsplat, jnp.maximum(tbs - 1, 0)])
                td = plsc.load_gather(
                    tail_v, [dsplat, jnp.maximum(tbd - 1, 0)])
                s_fin = jnp.where(tbs > 0, ts, sv)
                d_fin = jnp.where(tbd > 0, td, dv)
                df = s_fin - d_fin
                acc[sl] = acc[sl] + df * df
                return c2

            lax.fori_loop(0, n_grp, group, 0)
            return carry

        lax.fori_loop(0, d_per_sc, column, 0)
        wait_stage()

        pltpu.sync_copy(acc, part_hbm.at[cid, pl.ds(tid * b_per_t, b_per_t)])

    return main_kernel


@functools.lru_cache(maxsize=None)
def _build_combine(B: int):
    NW = NC * NS
    b_per_w = B // NW

    @functools.partial(
        pl.kernel,
        out_type=jax.ShapeDtypeStruct((B,), jnp.float32),
        mesh=_mesh(),
        scratch_types=[
            pltpu.VMEM((b_per_w,), jnp.float32),
            pltpu.VMEM((b_per_w,), jnp.float32),
            pltpu.VMEM((b_per_w,), jnp.float32),
        ],
        compiler_params=pltpu.CompilerParams(needs_layout_passes=False),
    )
    def combine_kernel(part_hbm, out_hbm, a0, a1, ob):
        wid = lax.axis_index("s") * NC + lax.axis_index("c")
        base = wid * b_per_w
        pltpu.sync_copy(part_hbm.at[0, pl.ds(base, b_per_w)], a0)
        pltpu.sync_copy(part_hbm.at[1, pl.ds(base, b_per_w)], a1)

        def group(i, carry):
            sl = pl.ds(i * L, L)
            x = a0[sl] + a1[sl] + jnp.float32(1e-12)
            iv = plsc.bitcast(x, jnp.int32)
            r = plsc.bitcast(jnp.int32(0x5F3759DF) - (iv >> 1), jnp.float32)
            half_x = jnp.float32(0.5) * x
            for _ in range(3):
                r = r * (jnp.float32(1.5) - half_x * r * r)
            ob[sl] = x * r
            return carry

        lax.fori_loop(0, b_per_w // L, group, 0)
        pltpu.sync_copy(ob, out_hbm.at[pl.ds(base, b_per_w)])

    return combine_kernel


def kernel(input_triplet, table):
    B = input_triplet.shape[0]
    V, D = table.shape
    src = input_triplet[:, 0].astype(jnp.int32)
    dst = input_triplet[:, 1].astype(jnp.int32)
    tabT = table.T
    tail = jnp.pad(table[MAINT:, :].T, ((0, 0), (0, TAILW - (V - MAINT))))
    part = _build_main(B, D, V)(src, dst, tabT, tail)
    return _build_combine(B)(part)
